# baseline (speedup 1.0000x reference)
import jax
import jax.numpy as jnp
from jax import lax
from jax.experimental import pallas as pl
from jax.experimental.pallas import tpu as pltpu

N_GROUP = 8
TOPK_GROUP = 4
TOP_K = 8
ROUTED_SCALING_FACTOR = 2.5


def _gate_chunk(lt, b):
    E, C = lt.shape
    G = E // N_GROUP

    sig = jax.nn.sigmoid(lt)
    swb = sig + b

    neg = jnp.float32(-jnp.inf)

    x = swb.reshape(N_GROUP, G, C)
    a1 = x
    a2 = jnp.full_like(x, neg)
    for k in (1, 2, 4):
        r1 = pltpu.roll(a1, k, axis=1)
        r2 = pltpu.roll(a2, k, axis=1)
        mn = jnp.minimum(a1, r1)
        a1 = jnp.maximum(a1, r1)
        a2 = jnp.maximum(jnp.maximum(a2, r2), mn)
    gsc = a1[:, 0, :] + a2[:, 0, :]

    iota_ng = lax.broadcasted_iota(jnp.int32, (N_GROUP, C), 0)
    cur = gsc
    for _ in range(TOPK_GROUP):
        m = jnp.max(cur, axis=0, keepdims=True)
        eq = cur == m
        fi = jnp.min(jnp.where(eq, iota_ng, N_GROUP), axis=0, keepdims=True)
        cur = jnp.where(iota_ng == fi, neg, cur)
    gmask = cur == neg

    t = jnp.where(
        jnp.broadcast_to(gmask[:, None, :], (N_GROUP, G, C)), x, 0.0
    ).reshape(E, C)

    H = E // 2
    lo = t[:H, :]
    hi = t[H:, :]
    lowcur0 = lo >= hi
    tc = jnp.maximum(lo, hi)
    tmin = jnp.minimum(lo, hi)
    iota_h = lax.broadcasted_iota(jnp.int32, (H, C), 0)
    eidx = jnp.where(lowcur0, iota_h, iota_h + H)
    for _ in range(TOP_K):
        m = jnp.max(tc, axis=0, keepdims=True)
        eq = tc == m
        fi = jnp.min(jnp.where(eq, eidx, E), axis=0, keepdims=True)
        sel = eidx == fi
        tc_new = jnp.where(sel, tmin, tc)
        tmin = jnp.where(sel, neg, tmin)
        tc = tc_new
        eidx = jnp.where(sel, eidx ^ H, eidx)
    ex1 = tmin == neg
    ex2 = tc == neg
    nml = (ex1 & lowcur0) | ex2
    nmh = (ex1 & (~lowcur0)) | ex2
    nm = jnp.concatenate([nml, nmh], axis=0)

    out = jnp.where(nm, sig, 0.0)
    s = jnp.sum(out, axis=0, keepdims=True) + 1e-20
    return out * (ROUTED_SCALING_FACTOR / s)


def _gate_block(h_ref, w_ref, b_ref, o_ref):
    B = h_ref.shape[0]

    bias = b_ref[...]
    w = w_ref[...]

    C = min(512, B)
    for c0 in range(0, B, C):
        lt = lax.dot_general(w, h_ref[c0:c0 + C, :], (((0,), (1,)), ((), ())),
                             preferred_element_type=jnp.float32)
        o_ref[c0:c0 + C, :] = _gate_chunk(lt, bias).T


@jax.jit
def kernel(hidden_states, weight, e_score_correction_bias):
    T, H = hidden_states.shape
    E = weight.shape[0]
    B = T
    for cand in (1024, 512, 256, 128, 64, 32, 16, 8):
        if T % cand == 0:
            B = cand
            break

    wT = weight.T
    bias = e_score_correction_bias.reshape(E, 1).astype(jnp.float32)

    return pl.pallas_call(
        _gate_block,
        grid=(T // B,),
        in_specs=[
            pl.BlockSpec((B, H), lambda i: (i, 0)),
            pl.BlockSpec((H, E), lambda i: (0, 0)),
            pl.BlockSpec((E, 1), lambda i: (0, 0)),
        ],
        out_specs=pl.BlockSpec((B, E), lambda i: (i, 0)),
        out_shape=jax.ShapeDtypeStruct((T, E), jnp.float32),
    )(hidden_states.astype(jnp.float32), wT, bias)


# speedup vs baseline: 1.0968x; 1.0968x over previous
import jax
import jax.numpy as jnp
from jax import lax
from jax.experimental import pallas as pl
from jax.experimental.pallas import tpu as pltpu

N_GROUP = 8
TOPK_GROUP = 4
TOP_K = 8
ROUTED_SCALING_FACTOR = 2.5


def _gate_chunk(lt, b):
    E, C = lt.shape
    G = E // N_GROUP

    sig = jax.nn.sigmoid(lt)
    swb = sig + b

    neg = jnp.float32(-jnp.inf)

    x = swb.reshape(N_GROUP, G, C)
    a1 = x
    a2 = jnp.full_like(x, neg)
    for k in (1, 2, 4):
        r1 = pltpu.roll(a1, k, axis=1)
        r2 = pltpu.roll(a2, k, axis=1)
        mn = jnp.minimum(a1, r1)
        a1 = jnp.maximum(a1, r1)
        a2 = jnp.maximum(jnp.maximum(a2, r2), mn)
    gsc = a1[:, 0, :] + a2[:, 0, :]

    iota_ng = lax.broadcasted_iota(jnp.int32, (N_GROUP, C), 0)
    cur = gsc
    for _ in range(TOPK_GROUP):
        m = jnp.max(cur, axis=0, keepdims=True)
        eq = cur == m
        fi = jnp.min(jnp.where(eq, iota_ng, N_GROUP), axis=0, keepdims=True)
        cur = jnp.where(iota_ng == fi, neg, cur)
    gmask = cur == neg

    t = jnp.where(
        jnp.broadcast_to(gmask[:, None, :], (N_GROUP, G, C)), x, 0.0
    ).reshape(E, C)

    H = E // 2
    lo = t[:H, :]
    hi = t[H:, :]
    lowcur0 = lo >= hi
    tc = jnp.maximum(lo, hi)
    tmin = jnp.minimum(lo, hi)
    iota_h = lax.broadcasted_iota(jnp.int32, (H, C), 0)
    eidx = jnp.where(lowcur0, iota_h, iota_h + H)
    for _ in range(TOP_K):
        m = jnp.max(tc, axis=0, keepdims=True)
        eq = tc == m
        fi = jnp.min(jnp.where(eq, eidx, E), axis=0, keepdims=True)
        sel = eidx == fi
        tc_new = jnp.where(sel, tmin, tc)
        tmin = jnp.where(sel, neg, tmin)
        tc = tc_new
        eidx = jnp.where(sel, eidx ^ H, eidx)
    ex1 = tmin == neg
    ex2 = tc == neg
    nml = (ex1 & lowcur0) | ex2
    nmh = (ex1 & (~lowcur0)) | ex2
    nm = jnp.concatenate([nml, nmh], axis=0)

    out = jnp.where(nm, sig, 0.0)
    s = jnp.sum(out, axis=0, keepdims=True) + 1e-20
    return out * (ROUTED_SCALING_FACTOR / s)


def _gate_block(h_ref, w_ref, b_ref, o_ref):
    B = h_ref.shape[0]

    bias = b_ref[...]
    w = w_ref[...]

    C = min(512, B)
    for c0 in range(0, B, C):
        lt = lax.dot_general(w, h_ref[c0:c0 + C, :], (((0,), (1,)), ((), ())),
                             preferred_element_type=jnp.float32)
        o_ref[c0:c0 + C, :] = _gate_chunk(lt, bias).T


@jax.jit
def kernel(hidden_states, weight, e_score_correction_bias):
    T, H = hidden_states.shape
    E = weight.shape[0]
    B = T
    for cand in (2048, 1024, 512, 256, 128, 64, 32, 16, 8):
        if T % cand == 0:
            B = cand
            break

    wT = weight.T
    bias = e_score_correction_bias.reshape(E, 1).astype(jnp.float32)

    return pl.pallas_call(
        _gate_block,
        grid=(T // B,),
        in_specs=[
            pl.BlockSpec((B, H), lambda i: (i, 0)),
            pl.BlockSpec((H, E), lambda i: (0, 0)),
            pl.BlockSpec((E, 1), lambda i: (0, 0)),
        ],
        out_specs=pl.BlockSpec((B, E), lambda i: (i, 0)),
        out_shape=jax.ShapeDtypeStruct((T, E), jnp.float32),
    )(hidden_states.astype(jnp.float32), wT, bias)
